# baseline (speedup 1.0000x reference)
import jax
import jax.numpy as jnp
from jax import lax
from jax.experimental import pallas as pl
from jax.experimental.pallas import tpu as pltpu
from jax.experimental.pallas import tpu_sc as plsc

_BATCH = 16384
_EMBED = 16
_NC = 2
_NS = 16
_NW = _NC * _NS
_BPW = _BATCH // _NW
_LANES = 16
_RING = 8


def _bias_body(uidx_hbm, iidx_hbm, ubias_hbm, ibias_hbm, bsum_hbm,
               idx_u, idx_i, bu, bi, bsum, sem):
    wid = lax.axis_index("s") * _NC + lax.axis_index("c")
    base = wid * _BPW
    pltpu.sync_copy(uidx_hbm.at[pl.ds(base, _BPW)], idx_u)
    pltpu.sync_copy(iidx_hbm.at[pl.ds(base, _BPW)], idx_i)
    cp_u = pltpu.async_copy(ubias_hbm.at[idx_u], bu, sem)
    cp_i = pltpu.async_copy(ibias_hbm.at[idx_i], bi, sem)
    cp_u.wait()
    cp_i.wait()
    for j in range(_BPW // _LANES):
        sl = pl.ds(j * _LANES, _LANES)
        bsum[sl] = bu[sl] + bi[sl]
    pltpu.sync_copy(bsum, bsum_hbm.at[pl.ds(base, _BPW)])


_bias_call = pl.kernel(
    _bias_body,
    out_type=jax.ShapeDtypeStruct((_BATCH,), jnp.float32),
    mesh=plsc.VectorSubcoreMesh(core_axis_name="c", subcore_axis_name="s"),
    compiler_params=pltpu.CompilerParams(use_tc_tiling_on_sc=False),
    scratch_types=[
        pltpu.VMEM((_BPW,), jnp.int32),
        pltpu.VMEM((_BPW,), jnp.int32),
        pltpu.VMEM((_BPW,), jnp.float32),
        pltpu.VMEM((_BPW,), jnp.float32),
        pltpu.VMEM((_BPW,), jnp.float32),
        pltpu.SemaphoreType.DMA,
    ],
)


_NFULL = (1000000 // 128) * 128
_AUXBASE = 1000000 - 128


_NCHUNK = _BPW // _LANES


def _dot_body(uidx_hbm, iidx_hbm, utT_hbm, itT_hbm, auxu_hbm, auxi_hbm,
              ubias_hbm, ibias_hbm,
              partials_hbm, bsum_hbm,
              idx_uv, idx_iv, ring_u, ring_i, aux_u, aux_i, bu, bi, bsum,
              pvec, semau, semai, sembu, sembi, bsem):
    wid = lax.axis_index("s") * _NC + lax.axis_index("c")
    base = wid * _BPW
    pltpu.sync_copy(uidx_hbm.at[pl.ds(base, _BPW)], idx_uv)
    pltpu.sync_copy(iidx_hbm.at[pl.ds(base, _BPW)], idx_iv)
    bcp_u = pltpu.async_copy(ubias_hbm.at[idx_uv], bu, bsem)
    bcp_i = pltpu.async_copy(ibias_hbm.at[idx_iv], bi, bsem)
    pltpu.sync_copy(auxu_hbm, aux_u)
    pltpu.sync_copy(auxi_hbm, aux_i)

    rows = jnp.arange(_LANES, dtype=jnp.int32)

    def fetch(tab_hbm, ring, sem, idx, r):
        b = pl.multiple_of((idx >> 7) * 128, 128)

        @pl.when(idx < _NFULL)
        def _():
            pltpu.async_copy(tab_hbm.at[:, pl.ds(b, 128)], ring.at[r], sem)

    def drain(tab_hbm, ring, sem, idx, r):
        @pl.when(idx < _NFULL)
        def _():
            pltpu.make_async_copy(tab_hbm.at[:, pl.ds(0, 128)], ring.at[r],
                                  sem).wait()

    def extract(ring, aux, idx, r):
        lane = jnp.full((_LANES,), idx & 127, jnp.int32)
        col = plsc.load_gather(ring.at[r], [rows, lane])
        alane = jnp.full((_LANES,), jnp.maximum(idx - _AUXBASE, 0), jnp.int32)
        acol = plsc.load_gather(aux, [rows, alane])
        return jnp.where(jnp.full((_LANES,), idx >= _NFULL), acol, col)

    def chunk_vecs(g):
        return (idx_uv[pl.ds(g * _LANES, _LANES)],
                idx_iv[pl.ds(g * _LANES, _LANES)])

    _H = _LANES // 2

    def issue_half(cu, ci, half):
        su, si = (semau, semai) if half == 0 else (sembu, sembi)
        for r in range(half * _H, half * _H + _H):
            fetch(utT_hbm, ring_u, su, cu[r], r)
            fetch(itT_hbm, ring_i, si, ci[r], r)

    def consume_half(cu, ci, half, acc):
        su, si = (semau, semai) if half == 0 else (sembu, sembi)
        for r in range(half * _H, half * _H + _H):
            drain(utT_hbm, ring_u, su, cu[r], r)
            drain(itT_hbm, ring_i, si, ci[r], r)
        for r in range(half * _H, half * _H + _H):
            ucol = extract(ring_u, aux_u, cu[r], r)
            vcol = extract(ring_i, aux_i, ci[r], r)
            acc = acc + ucol * vcol
        return acc

    cu0, ci0 = chunk_vecs(0)
    issue_half(cu0, ci0, 0)
    issue_half(cu0, ci0, 1)

    def block(g, carry):
        acc, cu, ci = carry
        nu, ni = chunk_vecs(jnp.minimum(g + 1, _NCHUNK - 1))
        acc = consume_half(cu, ci, 0, acc)

        @pl.when(g + 1 < _NCHUNK)
        def _():
            issue_half(nu, ni, 0)

        acc = consume_half(cu, ci, 1, acc)

        @pl.when(g + 1 < _NCHUNK)
        def _():
            issue_half(nu, ni, 1)

        return acc, nu, ni

    acc, _, _ = lax.fori_loop(
        0, _NCHUNK, block,
        (jnp.zeros((_LANES,), jnp.float32), cu0, ci0))
    pvec[...] = acc
    pltpu.sync_copy(pvec, partials_hbm.at[pl.ds(wid * _LANES, _LANES)])

    bcp_u.wait()
    bcp_i.wait()
    for j in range(_BPW // _LANES):
        sl = pl.ds(j * _LANES, _LANES)
        bsum[sl] = bu[sl] + bi[sl]
    pltpu.sync_copy(bsum, bsum_hbm.at[pl.ds(base, _BPW)])


_dot_call = pl.kernel(
    _dot_body,
    out_type=(
        jax.ShapeDtypeStruct((_NW * _LANES,), jnp.float32),
        jax.ShapeDtypeStruct((_BATCH,), jnp.float32),
    ),
    mesh=plsc.VectorSubcoreMesh(core_axis_name="c", subcore_axis_name="s"),
    compiler_params=pltpu.CompilerParams(use_tc_tiling_on_sc=True,
                                         needs_layout_passes=False),
    scratch_types=[
        pltpu.VMEM((_BPW,), jnp.int32),
        pltpu.VMEM((_BPW,), jnp.int32),
        pltpu.VMEM((_LANES, _EMBED, 128), jnp.float32),
        pltpu.VMEM((_LANES, _EMBED, 128), jnp.float32),
        pltpu.VMEM((_EMBED, 128), jnp.float32),
        pltpu.VMEM((_EMBED, 128), jnp.float32),
        pltpu.VMEM((_BPW,), jnp.float32),
        pltpu.VMEM((_BPW,), jnp.float32),
        pltpu.VMEM((_BPW,), jnp.float32),
        pltpu.VMEM((_LANES,), jnp.float32),
        pltpu.SemaphoreType.DMA,
        pltpu.SemaphoreType.DMA,
        pltpu.SemaphoreType.DMA,
        pltpu.SemaphoreType.DMA,
        pltpu.SemaphoreType.DMA,
    ],
)


def _tc_body(partials_ref, bsum_ref, out_ref):
    dot = jnp.sum(partials_ref[...])
    out_ref[...] = jnp.maximum(bsum_ref[...] + dot, 0.0)


def kernel(inputs, user_table, user_bias_table, item_table, item_bias_table):
    user_idx = inputs[:, 0]
    item_idx = inputs[:, 1]
    partials, bsum = _dot_call(
        user_idx, item_idx, user_table.T, item_table.T,
        user_table[_AUXBASE:].T, item_table[_AUXBASE:].T,
        user_bias_table.reshape(-1), item_bias_table.reshape(-1))
    out = pl.pallas_call(
        _tc_body,
        out_shape=jax.ShapeDtypeStruct((128, 128), jnp.float32),
    )(partials.reshape(4, 128), bsum.reshape(128, 128))
    return out.reshape(_BATCH, 1)


# speedup vs baseline: 1.5648x; 1.5648x over previous
import jax
import jax.numpy as jnp
from jax import lax
from jax.experimental import pallas as pl
from jax.experimental.pallas import tpu as pltpu
from jax.experimental.pallas import tpu_sc as plsc

_BATCH = 16384
_EMBED = 16
_NC = 2
_NS = 16
_NW = _NC * _NS
_BPW = _BATCH // _NW
_LANES = 16
_RING = 8


def _bias_body(uidx_hbm, iidx_hbm, ubias_hbm, ibias_hbm, bsum_hbm,
               idx_u, idx_i, bu, bi, bsum, sem):
    wid = lax.axis_index("s") * _NC + lax.axis_index("c")
    base = wid * _BPW
    pltpu.sync_copy(uidx_hbm.at[pl.ds(base, _BPW)], idx_u)
    pltpu.sync_copy(iidx_hbm.at[pl.ds(base, _BPW)], idx_i)
    cp_u = pltpu.async_copy(ubias_hbm.at[idx_u], bu, sem)
    cp_i = pltpu.async_copy(ibias_hbm.at[idx_i], bi, sem)
    cp_u.wait()
    cp_i.wait()
    for j in range(_BPW // _LANES):
        sl = pl.ds(j * _LANES, _LANES)
        bsum[sl] = bu[sl] + bi[sl]
    pltpu.sync_copy(bsum, bsum_hbm.at[pl.ds(base, _BPW)])


_bias_call = pl.kernel(
    _bias_body,
    out_type=jax.ShapeDtypeStruct((_BATCH,), jnp.float32),
    mesh=plsc.VectorSubcoreMesh(core_axis_name="c", subcore_axis_name="s"),
    compiler_params=pltpu.CompilerParams(use_tc_tiling_on_sc=False),
    scratch_types=[
        pltpu.VMEM((_BPW,), jnp.int32),
        pltpu.VMEM((_BPW,), jnp.int32),
        pltpu.VMEM((_BPW,), jnp.float32),
        pltpu.VMEM((_BPW,), jnp.float32),
        pltpu.VMEM((_BPW,), jnp.float32),
        pltpu.SemaphoreType.DMA,
    ],
)


_NFULL = (1000000 // 128) * 128
_AUXBASE = 1000000 - 128


_NCHUNK = _BPW // _LANES


def _dot_body(uidx_hbm, iidx_hbm, utT_hbm, itT_hbm, auxu_hbm, auxi_hbm,
              partials_hbm,
              idx_uv, idx_iv, ring_u, ring_i, aux_u, aux_i, pvec,
              semau, semai, sembu, sembi, semcu, semci):
    wid = lax.axis_index("s") * _NC + lax.axis_index("c")
    base = wid * _BPW
    pltpu.sync_copy(uidx_hbm.at[pl.ds(base, _BPW)], idx_uv.at[pl.ds(0, _BPW)])
    pltpu.sync_copy(iidx_hbm.at[pl.ds(base, _BPW)], idx_iv.at[pl.ds(0, _BPW)])
    pltpu.sync_copy(auxu_hbm, aux_u)
    pltpu.sync_copy(auxi_hbm, aux_i)

    rows = jnp.arange(_LANES, dtype=jnp.int32)

    def fetch(tab_hbm, ring, sem, idx, r):
        b = pl.multiple_of((idx >> 7) * 128, 128)

        @pl.when(idx < _NFULL)
        def _():
            pltpu.async_copy(tab_hbm.at[:, pl.ds(b, 128)], ring.at[r], sem)

    def drain(tab_hbm, ring, sem, idx, r):
        @pl.when(idx < _NFULL)
        def _():
            pltpu.make_async_copy(tab_hbm.at[:, pl.ds(0, 128)], ring.at[r],
                                  sem).wait()

    def extract(ring, aux, idx, r):
        lane = jnp.full((_LANES,), idx & 127, jnp.int32)
        col = plsc.load_gather(ring.at[r], [rows, lane])
        alane = jnp.full((_LANES,), jnp.maximum(idx - _AUXBASE, 0), jnp.int32)
        acol = plsc.load_gather(aux, [rows, alane])
        return jnp.where(jnp.full((_LANES,), idx >= _NFULL), acol, col)

    _H = 8
    _NH = _BPW // _H
    sems = ((semau, semai), (sembu, sembi), (semcu, semci))

    def half_vecs(h):
        return idx_uv[pl.ds(h * _H, _LANES)], idx_iv[pl.ds(h * _H, _LANES)]

    def issue_half(h, k):
        su, si = sems[k]
        cu, ci = half_vecs(h)
        for r in range(_H):
            fetch(utT_hbm, ring_u.at[k], su, cu[r], r)
            fetch(itT_hbm, ring_i.at[k], si, ci[r], r)

    def consume_half(h, k, acc):
        su, si = sems[k]
        cu, ci = half_vecs(h)
        for r in range(_H):
            drain(utT_hbm, ring_u.at[k], su, cu[r], r)
            drain(itT_hbm, ring_i.at[k], si, ci[r], r)
        for r in range(_H):
            ucol = extract(ring_u.at[k], aux_u, cu[r], r)
            vcol = extract(ring_i.at[k], aux_i, ci[r], r)
            acc = acc + ucol * vcol
        return acc

    issue_half(0, 0)
    issue_half(1, 1)

    def block(g, acc):
        for k in range(3):
            h = 3 * g + k
            acc = consume_half(h, k, acc)

            @pl.when(h + 2 < _NH)
            def _():
                issue_half(h + 2, (k + 2) % 3)

        return acc

    acc = lax.fori_loop(0, (_NH - 1) // 3, block,
                        jnp.zeros((_LANES,), jnp.float32))
    acc = consume_half(_NH - 1, (_NH - 1) % 3, acc)
    pvec[...] = acc
    pltpu.sync_copy(pvec, partials_hbm.at[pl.ds(wid * _LANES, _LANES)])


_dot_call = pl.kernel(
    _dot_body,
    out_type=jax.ShapeDtypeStruct((_NW * _LANES,), jnp.float32),
    mesh=plsc.VectorSubcoreMesh(core_axis_name="c", subcore_axis_name="s"),
    compiler_params=pltpu.CompilerParams(use_tc_tiling_on_sc=True,
                                         needs_layout_passes=False),
    scratch_types=[
        pltpu.VMEM((_BPW + _LANES,), jnp.int32),
        pltpu.VMEM((_BPW + _LANES,), jnp.int32),
        pltpu.VMEM((3, 8, _EMBED, 128), jnp.float32),
        pltpu.VMEM((3, 8, _EMBED, 128), jnp.float32),
        pltpu.VMEM((_EMBED, 128), jnp.float32),
        pltpu.VMEM((_EMBED, 128), jnp.float32),
        pltpu.VMEM((_LANES,), jnp.float32),
        pltpu.SemaphoreType.DMA,
        pltpu.SemaphoreType.DMA,
        pltpu.SemaphoreType.DMA,
        pltpu.SemaphoreType.DMA,
        pltpu.SemaphoreType.DMA,
        pltpu.SemaphoreType.DMA,
    ],
)


def _tc_body(partials_ref, bsum_ref, out_ref):
    dot = jnp.sum(partials_ref[...])
    out_ref[...] = jnp.maximum(bsum_ref[...] + dot, 0.0)


def kernel(inputs, user_table, user_bias_table, item_table, item_bias_table):
    user_idx = inputs[:, 0]
    item_idx = inputs[:, 1]
    bsum = _bias_call(user_idx, item_idx, user_bias_table.reshape(-1),
                      item_bias_table.reshape(-1))
    partials = _dot_call(user_idx, item_idx, user_table.T, item_table.T,
                         user_table[_AUXBASE:].T, item_table[_AUXBASE:].T)
    out = pl.pallas_call(
        _tc_body,
        out_shape=jax.ShapeDtypeStruct((128, 128), jnp.float32),
    )(partials.reshape(4, 128), bsum.reshape(128, 128))
    return out.reshape(_BATCH, 1)


# speedup vs baseline: 1.5799x; 1.0097x over previous
import jax
import jax.numpy as jnp
from jax import lax
from jax.experimental import pallas as pl
from jax.experimental.pallas import tpu as pltpu
from jax.experimental.pallas import tpu_sc as plsc

_BATCH = 16384
_EMBED = 16
_NC = 2
_NS = 16
_NW = _NC * _NS
_BPW = _BATCH // _NW
_LANES = 16
_RING = 8


def _finish_body(uidx_hbm, iidx_hbm, ubias_hbm, ibias_hbm, partials_hbm,
                 out_hbm,
                 idx_u, idx_i, bu, bi, part, res, sem):
    wid = lax.axis_index("s") * _NC + lax.axis_index("c")
    base = wid * _BPW
    pltpu.sync_copy(uidx_hbm.at[pl.ds(base, _BPW)], idx_u)
    pltpu.sync_copy(iidx_hbm.at[pl.ds(base, _BPW)], idx_i)
    cp_u = pltpu.async_copy(ubias_hbm.at[idx_u], bu, sem)
    cp_i = pltpu.async_copy(ibias_hbm.at[idx_i], bi, sem)
    pltpu.sync_copy(partials_hbm, part)

    vacc = jnp.zeros((_LANES,), jnp.float32)
    for j in range(_NW * _LANES // _LANES):
        vacc = vacc + part[pl.ds(j * _LANES, _LANES)]
    dotv = jnp.full((_LANES,), jnp.sum(vacc), jnp.float32)

    cp_u.wait()
    cp_i.wait()
    for j in range(_BPW // _LANES):
        sl = pl.ds(j * _LANES, _LANES)
        res[sl] = jnp.maximum(bu[sl] + bi[sl] + dotv, 0.0)
    pltpu.sync_copy(res, out_hbm.at[pl.ds(base, _BPW)])


_finish_call = pl.kernel(
    _finish_body,
    out_type=jax.ShapeDtypeStruct((_BATCH,), jnp.float32),
    mesh=plsc.VectorSubcoreMesh(core_axis_name="c", subcore_axis_name="s"),
    compiler_params=pltpu.CompilerParams(use_tc_tiling_on_sc=False,
                                         needs_layout_passes=False),
    scratch_types=[
        pltpu.VMEM((_BPW,), jnp.int32),
        pltpu.VMEM((_BPW,), jnp.int32),
        pltpu.VMEM((_BPW,), jnp.float32),
        pltpu.VMEM((_BPW,), jnp.float32),
        pltpu.VMEM((_NW * _LANES,), jnp.float32),
        pltpu.VMEM((_BPW,), jnp.float32),
        pltpu.SemaphoreType.DMA,
    ],
)


_NFULL = (1000000 // 128) * 128
_AUXBASE = 1000000 - 128


_NCHUNK = _BPW // _LANES


def _dot_body(uidx_hbm, iidx_hbm, utT_hbm, itT_hbm, auxu_hbm, auxi_hbm,
              partials_hbm,
              idx_uv, idx_iv, ring_u, ring_i, aux_u, aux_i, pvec,
              semau, semai, sembu, sembi, semcu, semci):
    wid = lax.axis_index("s") * _NC + lax.axis_index("c")
    base = wid * _BPW
    pltpu.sync_copy(uidx_hbm.at[pl.ds(base, _BPW)], idx_uv.at[pl.ds(0, _BPW)])
    pltpu.sync_copy(iidx_hbm.at[pl.ds(base, _BPW)], idx_iv.at[pl.ds(0, _BPW)])
    pltpu.sync_copy(auxu_hbm, aux_u)
    pltpu.sync_copy(auxi_hbm, aux_i)

    rows = jnp.arange(_LANES, dtype=jnp.int32)

    def fetch(tab_hbm, ring, sem, idx, r):
        b = pl.multiple_of((idx >> 7) * 128, 128)

        @pl.when(idx < _NFULL)
        def _():
            pltpu.async_copy(tab_hbm.at[:, pl.ds(b, 128)], ring.at[r], sem)

    def drain(tab_hbm, ring, sem, idx, r):
        @pl.when(idx < _NFULL)
        def _():
            pltpu.make_async_copy(tab_hbm.at[:, pl.ds(0, 128)], ring.at[r],
                                  sem).wait()

    def extract(ring, aux, idx, r):
        lane = jnp.full((_LANES,), idx & 127, jnp.int32)
        col = plsc.load_gather(ring.at[r], [rows, lane])
        alane = jnp.full((_LANES,), jnp.maximum(idx - _AUXBASE, 0), jnp.int32)
        acol = plsc.load_gather(aux, [rows, alane])
        return jnp.where(jnp.full((_LANES,), idx >= _NFULL), acol, col)

    _H = 8
    _NH = _BPW // _H
    sems = ((semau, semai), (sembu, sembi), (semcu, semci))

    def half_vecs(h):
        return idx_uv[pl.ds(h * _H, _LANES)], idx_iv[pl.ds(h * _H, _LANES)]

    def issue_half(h, k):
        su, si = sems[k]
        cu, ci = half_vecs(h)
        for r in range(_H):
            fetch(utT_hbm, ring_u.at[k], su, cu[r], r)
            fetch(itT_hbm, ring_i.at[k], si, ci[r], r)

    def consume_half(h, k, acc):
        su, si = sems[k]
        cu, ci = half_vecs(h)
        for r in range(_H):
            drain(utT_hbm, ring_u.at[k], su, cu[r], r)
            drain(itT_hbm, ring_i.at[k], si, ci[r], r)
        for r in range(_H):
            ucol = extract(ring_u.at[k], aux_u, cu[r], r)
            vcol = extract(ring_i.at[k], aux_i, ci[r], r)
            acc = acc + ucol * vcol
        return acc

    issue_half(0, 0)
    issue_half(1, 1)

    def block(g, acc):
        for k in range(3):
            h = 3 * g + k
            acc = consume_half(h, k, acc)

            @pl.when(h + 2 < _NH)
            def _():
                issue_half(h + 2, (k + 2) % 3)

        return acc

    acc = lax.fori_loop(0, (_NH - 1) // 3, block,
                        jnp.zeros((_LANES,), jnp.float32))
    acc = consume_half(_NH - 1, (_NH - 1) % 3, acc)
    pvec[...] = acc
    pltpu.sync_copy(pvec, partials_hbm.at[pl.ds(wid * _LANES, _LANES)])


_dot_call = pl.kernel(
    _dot_body,
    out_type=jax.ShapeDtypeStruct((_NW * _LANES,), jnp.float32),
    mesh=plsc.VectorSubcoreMesh(core_axis_name="c", subcore_axis_name="s"),
    compiler_params=pltpu.CompilerParams(use_tc_tiling_on_sc=True,
                                         needs_layout_passes=False),
    scratch_types=[
        pltpu.VMEM((_BPW + _LANES,), jnp.int32),
        pltpu.VMEM((_BPW + _LANES,), jnp.int32),
        pltpu.VMEM((3, 8, _EMBED, 128), jnp.float32),
        pltpu.VMEM((3, 8, _EMBED, 128), jnp.float32),
        pltpu.VMEM((_EMBED, 128), jnp.float32),
        pltpu.VMEM((_EMBED, 128), jnp.float32),
        pltpu.VMEM((_LANES,), jnp.float32),
        pltpu.SemaphoreType.DMA,
        pltpu.SemaphoreType.DMA,
        pltpu.SemaphoreType.DMA,
        pltpu.SemaphoreType.DMA,
        pltpu.SemaphoreType.DMA,
        pltpu.SemaphoreType.DMA,
    ],
)


def kernel(inputs, user_table, user_bias_table, item_table, item_bias_table):
    user_idx = inputs[:, 0]
    item_idx = inputs[:, 1]
    partials = _dot_call(user_idx, item_idx, user_table.T, item_table.T,
                         user_table[_AUXBASE:].T, item_table[_AUXBASE:].T)
    out = _finish_call(user_idx, item_idx, user_bias_table.reshape(-1),
                       item_bias_table.reshape(-1), partials)
    return out.reshape(_BATCH, 1)
